# baseline (speedup 1.0000x reference)
import functools

import jax
import jax.numpy as jnp
from jax import lax
from jax.experimental import pallas as pl
from jax.experimental.pallas import tpu as pltpu
from jax.experimental.pallas import tpu_sc as plsc

B = 16384
NF = 16
NW = 32
BPW = B // NW
CHUNK = 128
NCHUNK = BPW // CHUNK


def _sc_factorization(uidx, sidx, ufac, sfac, uoff, soff, goff16):
    mesh = plsc.VectorSubcoreMesh(core_axis_name="c", subcore_axis_name="s")

    @functools.partial(
        pl.kernel,
        mesh=mesh,
        out_type=jax.ShapeDtypeStruct((B,), jnp.float32),
        compiler_params=pltpu.CompilerParams(
            needs_layout_passes=False, use_tc_tiling_on_sc=False),
        scratch_types=[
            pltpu.VMEM((NCHUNK, CHUNK), jnp.int32),
            pltpu.VMEM((NCHUNK, CHUNK), jnp.int32),
            pltpu.VMEM((BPW, NF), jnp.float32),
            pltpu.VMEM((BPW, NF), jnp.float32),
            pltpu.VMEM((BPW,), jnp.float32),
            pltpu.VMEM((BPW,), jnp.float32),
            pltpu.VMEM((16,), jnp.float32),
            pltpu.VMEM((BPW,), jnp.float32),
            pltpu.SemaphoreType.DMA,
        ],
    )
    def k(uidx_hbm, sidx_hbm, ufac_hbm, sfac_hbm, uoff_hbm, soff_hbm,
          g_hbm, out_hbm, uidx_v, sidx_v, urows_v, srows_v, uoffs_v,
          soffs_v, g_v, out_v, sem):
        wid = lax.axis_index("s") * 2 + lax.axis_index("c")
        base = wid * BPW

        pltpu.sync_copy(uidx_hbm.at[wid], uidx_v)
        pltpu.sync_copy(sidx_hbm.at[wid], sidx_v)
        pltpu.sync_copy(g_hbm, g_v)

        copies = []
        for j in range(NCHUNK):
            dst = pl.ds(j * CHUNK, CHUNK)
            copies.append(
                pltpu.async_copy(ufac_hbm.at[uidx_v.at[j]], urows_v.at[dst], sem))
            copies.append(
                pltpu.async_copy(sfac_hbm.at[sidx_v.at[j]], srows_v.at[dst], sem))
            copies.append(
                pltpu.async_copy(uoff_hbm.at[uidx_v.at[j]], uoffs_v.at[dst], sem))
            copies.append(
                pltpu.async_copy(soff_hbm.at[sidx_v.at[j]], soffs_v.at[dst], sem))
        for c in copies:
            c.wait()

        g = g_v[...]

        def body(c, carry):
            rows = jnp.arange(16, dtype=jnp.int32) + c * 16
            acc = g
            for j in range(NF):
                cols = jnp.full((16,), j, dtype=jnp.int32)
                uv = plsc.load_gather(urows_v, [rows, cols])
                sv = plsc.load_gather(srows_v, [rows, cols])
                acc = acc + uv * sv
            sl = pl.ds(c * 16, 16)
            out_v[sl] = acc + uoffs_v[sl] + soffs_v[sl]
            return carry

        lax.fori_loop(0, BPW // 16, body, 0)

        pltpu.sync_copy(out_v, out_hbm.at[pl.ds(base, BPW)])

    return k(uidx, sidx, ufac, sfac, uoff, soff, goff16)


def kernel(user_indexes, statement_indexes, user_factors, statement_factors,
           user_offsets, statement_offsets, global_offset):
    uidx = user_indexes.astype(jnp.int32).reshape(NW, NCHUNK, CHUNK)
    sidx = statement_indexes.astype(jnp.int32).reshape(NW, NCHUNK, CHUNK)
    uoff = user_offsets.reshape(-1)
    soff = statement_offsets.reshape(-1)
    g16 = jnp.broadcast_to(global_offset.reshape(1), (16,))
    return _sc_factorization(uidx, sidx, user_factors, statement_factors,
                             uoff, soff, g16)
